# baseline (speedup 1.0000x reference)
import jax
import jax.numpy as jnp
from jax.experimental import pallas as pl
from jax.experimental.pallas import tpu as pltpu

E = 64
TOP_K = 2
D = 768
F = 1024
S = 2048
BLK = 128


def _router_kernel(h_ref, gw_ref, idx_ref, wgt_ref, meta_ref):
    h = h_ref[...]
    gw = gw_ref[...]
    logits = jax.lax.dot_general(h, gw, (((1,), (1,)), ((), ())),
                                 preferred_element_type=jnp.float32)
    p = jax.nn.softmax(logits, axis=-1)

    lane = jax.lax.broadcasted_iota(jnp.int32, (S, E), 1)
    m0 = jnp.max(p, axis=-1, keepdims=True)
    e0 = jnp.min(jnp.where(p == m0, lane, E), axis=-1)
    p_masked = jnp.where(lane == e0[:, None], -1.0, p)
    m1 = jnp.max(p_masked, axis=-1, keepdims=True)
    e1 = jnp.min(jnp.where(p_masked == m1, lane, E), axis=-1)
    p0 = m0[:, 0]
    p1 = m1[:, 0]
    denom = p0 + p1
    w0 = p0 / denom
    w1 = p1 / denom

    oh0 = (lane == e0[:, None]).astype(jnp.float32)
    oh1 = (lane == e1[:, None]).astype(jnp.float32)
    occ = oh0 + oh1
    ti = jax.lax.broadcasted_iota(jnp.int32, (S, S), 0)
    tj = jax.lax.broadcasted_iota(jnp.int32, (S, S), 1)
    tril = (tj <= ti).astype(jnp.float32)
    csum = jax.lax.dot_general(tril, occ, (((1,), (0,)), ((), ())),
                               preferred_element_type=jnp.float32)
    r0 = jnp.sum(csum * oh0, axis=-1) - 1.0
    r1 = jnp.sum(csum * oh1, axis=-1) - 1.0

    counts = csum[S - 1, :]
    nsub = jnp.floor((counts + (BLK - 1)) / BLK)

    zi = jnp.zeros((S,), jnp.int32)
    idx_ref[...] = jnp.stack([e0, e1,
                              r0.astype(jnp.int32), r1.astype(jnp.int32),
                              zi, zi, zi, zi], axis=0)
    wz = jnp.zeros((S,), jnp.float32)
    wgt_ref[...] = jnp.stack([w0, w1, wz, wz, wz, wz, wz, wz], axis=0)
    nsub_p = jnp.concatenate([nsub.astype(jnp.int32), zi[:E]])
    mz = jnp.zeros((128,), jnp.int32)
    meta_ref[...] = jnp.stack([nsub_p, mz, mz, mz, mz, mz, mz, mz], axis=0)


def _moe_kernel(nsub_ref, h_ref, idx_ref, wgt_ref,
                wg_ref, wu_ref, wd_ref, out_ref):
    i = pl.program_id(0)
    nsub = nsub_ref[i]

    @pl.when(i == 0)
    def _init():
        out_ref[...] = jnp.zeros_like(out_ref)

    ids = idx_ref[...]
    wts = wgt_ref[...]
    e0 = ids[0:1, :]
    e1 = ids[1:2, :]
    r0 = ids[2:3, :]
    r1 = ids[3:4, :]
    w0 = wts[0:1, :]
    w1 = wts[1:2, :]
    jrow = jax.lax.broadcasted_iota(jnp.int32, (BLK, S), 0)
    h = h_ref[...]

    def body(j, carry):
        sr = j * BLK
        m0 = (e0 == i) & ((r0 - sr) == jrow)
        m1 = (e1 == i) & ((r1 - sr) == jrow)
        disp = m0.astype(jnp.float32) + m1.astype(jnp.float32)
        x = jax.lax.dot_general(disp, h, (((1,), (0,)), ((), ())),
                                preferred_element_type=jnp.float32)
        g = jax.lax.dot_general(x, wg_ref[0], (((1,), (0,)), ((), ())),
                                preferred_element_type=jnp.float32)
        u = jax.lax.dot_general(x, wu_ref[0], (((1,), (0,)), ((), ())),
                                preferred_element_type=jnp.float32)
        a = g * jax.lax.logistic(g) * u
        y = jax.lax.dot_general(a, wd_ref[0], (((1,), (0,)), ((), ())),
                                preferred_element_type=jnp.float32)
        wdisp = m0.astype(jnp.float32) * w0 + m1.astype(jnp.float32) * w1
        out_ref[...] += jax.lax.dot_general(wdisp, y, (((0,), (0,)), ((), ())),
                                            preferred_element_type=jnp.float32)
        return carry

    jax.lax.fori_loop(0, nsub, body, 0)


@jax.jit
def kernel(hidden_states, gate_w, w_gate_proj, w_up_proj, w_down_proj):
    b, s, d = hidden_states.shape
    h = hidden_states.reshape(s, d)

    idx, wgt, meta = pl.pallas_call(
        _router_kernel,
        out_shape=(
            jax.ShapeDtypeStruct((8, S), jnp.int32),
            jax.ShapeDtypeStruct((8, S), jnp.float32),
            jax.ShapeDtypeStruct((8, 128), jnp.int32),
        ),
    )(h, gate_w)

    nsub = meta[0, :E]

    grid_spec = pltpu.PrefetchScalarGridSpec(
        num_scalar_prefetch=1,
        grid=(E,),
        in_specs=[
            pl.BlockSpec((S, D), lambda i, *_: (0, 0)),
            pl.BlockSpec((8, S), lambda i, *_: (0, 0)),
            pl.BlockSpec((8, S), lambda i, *_: (0, 0)),
            pl.BlockSpec((1, D, F), lambda i, ns: (i, 0, 0)),
            pl.BlockSpec((1, D, F), lambda i, ns: (i, 0, 0)),
            pl.BlockSpec((1, F, D), lambda i, ns: (i, 0, 0)),
        ],
        out_specs=pl.BlockSpec((S, D), lambda i, *_: (0, 0)),
    )
    out = pl.pallas_call(
        _moe_kernel,
        grid_spec=grid_spec,
        out_shape=jax.ShapeDtypeStruct((S, D), jnp.float32),
    )(nsub, h, idx, wgt, w_gate_proj, w_up_proj, w_down_proj)

    return out.reshape(b, s, d)
